# baseline (speedup 1.0000x reference)
import functools
import jax
import jax.numpy as jnp
from jax import lax
from jax.experimental import pallas as pl
from jax.experimental.pallas import tpu as pltpu

try:
    from jax.experimental.pallas import tpu_sc as plsc
except ImportError:
    plsc = None

TB = 1024
TBS = 256
NW = 32


def _silu(t):
    return t * jax.nn.sigmoid(t)


def _router_shared(xf_ref, sw1_ref, sw3_ref, sw2_ref, gw_ref,
                   bias_ref, shared_ref, idx_ref, wts_ref, *, n_shared,
                   n_routed):
    scores = jax.nn.sigmoid(
        jnp.dot(xf_ref[...], gw_ref[...], preferred_element_type=jnp.float32))
    sel = scores + bias_ref[...]
    e_iota = lax.broadcasted_iota(jnp.int32, sel.shape, 1)

    v0 = jnp.max(sel, axis=1, keepdims=True)
    idx0 = jnp.min(jnp.where(sel == v0, e_iota, n_routed), axis=1)
    sel2 = jnp.where(e_iota == idx0[:, None], -jnp.inf, sel)
    v1 = jnp.max(sel2, axis=1, keepdims=True)
    idx1 = jnp.min(jnp.where(sel2 == v1, e_iota, n_routed), axis=1)

    s0 = jnp.sum(jnp.where(e_iota == idx0[:, None], scores, 0.0), axis=1)
    s1 = jnp.sum(jnp.where(e_iota == idx1[:, None], scores, 0.0), axis=1)
    denom = s0 + s1
    idx_ref[:, 0] = idx0
    idx_ref[:, 1] = idx1
    wts_ref[:, 0] = s0 / denom
    wts_ref[:, 1] = s1 / denom

    xb = xf_ref[...].astype(jnp.bfloat16)
    acc = jnp.zeros_like(shared_ref)
    for e in range(n_shared):
        h = _silu(jnp.dot(xb, sw1_ref[e].astype(jnp.bfloat16),
                          preferred_element_type=jnp.float32))
        h = h * jnp.dot(xb, sw3_ref[e].astype(jnp.bfloat16),
                        preferred_element_type=jnp.float32)
        acc = acc + jnp.dot(h.astype(jnp.bfloat16),
                            sw2_ref[e].astype(jnp.bfloat16),
                            preferred_element_type=jnp.float32)
    shared_ref[...] = acc


def _sc_scatter_rows(xf, pos0, pos1):
    t, d = xf.shape
    tk = 2 * t
    rows_w = t // NW
    mesh = plsc.VectorSubcoreMesh(core_axis_name="c", subcore_axis_name="s")

    @functools.partial(
        pl.kernel, mesh=mesh,
        out_type=jax.ShapeDtypeStruct((tk, d), jnp.float32),
        scratch_types=[
            pltpu.VMEM((rows_w,), jnp.int32),
            pltpu.VMEM((rows_w, d), jnp.float32),
            pltpu.SemaphoreType.DMA,
        ],
    )
    def scat(xf_hbm, p0_hbm, p1_hbm, xs_hbm, idx_v, rows_v, sem):
        wid = lax.axis_index("s") * 2 + lax.axis_index("c")
        base = wid * rows_w
        pltpu.sync_copy(xf_hbm.at[pl.ds(base, rows_w)], rows_v)
        pltpu.sync_copy(p0_hbm.at[pl.ds(base, rows_w)], idx_v)
        pltpu.async_copy(rows_v, xs_hbm.at[idx_v], sem).wait()
        pltpu.sync_copy(p1_hbm.at[pl.ds(base, rows_w)], idx_v)
        pltpu.async_copy(rows_v, xs_hbm.at[idx_v], sem).wait()

    return scat(xf, pos0, pos1)


def _grouped_ffn(seg_ref, xs_ref, ws_ref, rw1_ref, rw3_ref, rw2_ref, y_ref,
                 *, n_routed):
    blk = pl.program_id(0)
    base = blk * TBS
    p_iota = base + lax.broadcasted_iota(jnp.int32, (TBS, 1), 0)
    xsb = xs_ref[...].astype(jnp.bfloat16)
    wcol = ws_ref[...]

    y_ref[...] = jnp.zeros_like(y_ref)
    for e in range(n_routed):
        s_e = seg_ref[e]
        e_e = seg_ref[n_routed + e]

        @pl.when((s_e < base + TBS) & (e_e > base))
        def _():
            m = (p_iota >= s_e) & (p_iota < e_e)
            he = _silu(jnp.dot(xsb, rw1_ref[e].astype(jnp.bfloat16),
                               preferred_element_type=jnp.float32))
            he = he * jnp.dot(xsb, rw3_ref[e].astype(jnp.bfloat16),
                              preferred_element_type=jnp.float32)
            he = jnp.where(m, wcol, 0.0) * he
            y_ref[...] += jnp.dot(he.astype(jnp.bfloat16),
                                  rw2_ref[e].astype(jnp.bfloat16),
                                  preferred_element_type=jnp.float32)


def _sc_combine(shared, y, pos0, pos1):
    t, d = shared.shape
    tok_w = t // NW
    nchunk = d // 16
    mesh = plsc.VectorSubcoreMesh(core_axis_name="c", subcore_axis_name="s")

    @functools.partial(
        pl.kernel, mesh=mesh,
        out_type=jax.ShapeDtypeStruct((t, d), jnp.float32),
        scratch_types=[
            pltpu.VMEM((tok_w,), jnp.int32),
            pltpu.VMEM((tok_w, d), jnp.float32),
            pltpu.VMEM((tok_w, d), jnp.float32),
            pltpu.SemaphoreType.DMA,
        ],
    )
    def comb(sh_hbm, y_hbm, p0_hbm, p1_hbm, out_hbm, idx_v, acc_v, g_v, sem):
        wid = lax.axis_index("s") * 2 + lax.axis_index("c")
        base = wid * tok_w
        pltpu.sync_copy(sh_hbm.at[pl.ds(base, tok_w)], acc_v)

        def accumulate():
            def row_body(r, carry):
                for c in range(nchunk):
                    sl = pl.ds(c * 16, 16)
                    acc_v[r, sl] = acc_v[r, sl] + g_v[r, sl]
                return carry
            lax.fori_loop(0, tok_w, row_body, 0)

        pltpu.sync_copy(p0_hbm.at[pl.ds(base, tok_w)], idx_v)
        pltpu.async_copy(y_hbm.at[idx_v], g_v, sem).wait()
        accumulate()
        pltpu.sync_copy(p1_hbm.at[pl.ds(base, tok_w)], idx_v)
        pltpu.async_copy(y_hbm.at[idx_v], g_v, sem).wait()
        accumulate()
        pltpu.sync_copy(acc_v, out_hbm.at[pl.ds(base, tok_w)])

    return comb(shared, y, pos0, pos1)


def kernel(x, shared_w1, shared_w3, shared_w2, routed_w1, routed_w3, routed_w2,
           gate_w, expert_bias):
    b, s, d = x.shape
    t = b * s
    tk = 2 * t
    n_routed = routed_w1.shape[0]
    xf = x.reshape(t, d)

    const = lambda i: (0, 0)
    const3 = lambda i: (0, 0, 0)

    shared, idx, wts = pl.pallas_call(
        functools.partial(_router_shared, n_shared=shared_w1.shape[0],
                          n_routed=n_routed),
        grid=(t // TB,),
        in_specs=[
            pl.BlockSpec((TB, d), lambda i: (i, 0)),
            pl.BlockSpec(shared_w1.shape, const3),
            pl.BlockSpec(shared_w3.shape, const3),
            pl.BlockSpec(shared_w2.shape, const3),
            pl.BlockSpec(gate_w.shape, const),
            pl.BlockSpec(expert_bias.shape, lambda i: (0,)),
        ],
        out_specs=[
            pl.BlockSpec((TB, d), lambda i: (i, 0)),
            pl.BlockSpec((TB, 2), lambda i: (i, 0)),
            pl.BlockSpec((TB, 2), lambda i: (i, 0)),
        ],
        out_shape=[
            jax.ShapeDtypeStruct((t, d), jnp.float32),
            jax.ShapeDtypeStruct((t, 2), jnp.int32),
            jax.ShapeDtypeStruct((t, 2), jnp.float32),
        ],
    )(xf, shared_w1, shared_w3, shared_w2, gate_w, expert_bias)

    idx0 = idx[:, 0]
    idx1 = idx[:, 1]
    e_ar = jnp.arange(n_routed, dtype=jnp.int32)
    oh = (idx0[:, None] == e_ar).astype(jnp.int32) + \
         (idx1[:, None] == e_ar).astype(jnp.int32)
    csum = jnp.cumsum(oh, axis=0)
    counts = csum[-1]
    cexcl = csum - oh
    start = jnp.concatenate([jnp.zeros((1,), jnp.int32),
                             jnp.cumsum(counts)[:-1].astype(jnp.int32)])
    seg = jnp.concatenate([start, (start + counts).astype(jnp.int32)])
    pos0 = start[idx0] + jnp.take_along_axis(
        cexcl, idx0[:, None], axis=1)[:, 0]
    pos1 = start[idx1] + jnp.take_along_axis(
        cexcl, idx1[:, None], axis=1)[:, 0] + (idx0 == idx1)
    pos0 = pos0.astype(jnp.int32)
    pos1 = pos1.astype(jnp.int32)
    w_sorted = jnp.zeros((tk, 1), jnp.float32)
    w_sorted = w_sorted.at[pos0, 0].set(wts[:, 0])
    w_sorted = w_sorted.at[pos1, 0].set(wts[:, 1])

    xs = _sc_scatter_rows(xf, pos0, pos1)

    y = pl.pallas_call(
        functools.partial(_grouped_ffn, n_routed=n_routed),
        grid_spec=pltpu.PrefetchScalarGridSpec(
            num_scalar_prefetch=1,
            grid=(tk // TBS,),
            in_specs=[
                pl.BlockSpec((TBS, d), lambda i, seg: (i, 0)),
                pl.BlockSpec((TBS, 1), lambda i, seg: (i, 0)),
                pl.BlockSpec(routed_w1.shape, lambda i, seg: (0, 0, 0)),
                pl.BlockSpec(routed_w3.shape, lambda i, seg: (0, 0, 0)),
                pl.BlockSpec(routed_w2.shape, lambda i, seg: (0, 0, 0)),
            ],
            out_specs=pl.BlockSpec((TBS, d), lambda i, seg: (i, 0)),
        ),
        out_shape=jax.ShapeDtypeStruct((tk, d), jnp.float32),
    )(seg, xs, w_sorted, routed_w1, routed_w3, routed_w2)

    out = _sc_combine(shared, y, pos0, pos1)

    return out.reshape(b, s, d), idx.reshape(b, s, 2)


# speedup vs baseline: 1.1388x; 1.1388x over previous
import functools
import jax
import jax.numpy as jnp
from jax import lax
from jax.experimental import pallas as pl
from jax.experimental.pallas import tpu as pltpu

try:
    from jax.experimental.pallas import tpu_sc as plsc
except ImportError:
    plsc = None

TB = 1024
TBS = 256
NW = 32


def _silu(t):
    return t * jax.nn.sigmoid(t)


def _router(xf_ref, gw_ref, bias_ref, idx_ref, wts_ref, *, n_routed):
    scores = jax.nn.sigmoid(
        jnp.dot(xf_ref[...], gw_ref[...], preferred_element_type=jnp.float32))
    sel = scores + bias_ref[...]
    e_iota = lax.broadcasted_iota(jnp.int32, sel.shape, 1)

    v0 = jnp.max(sel, axis=1, keepdims=True)
    idx0 = jnp.min(jnp.where(sel == v0, e_iota, n_routed), axis=1)
    sel2 = jnp.where(e_iota == idx0[:, None], -jnp.inf, sel)
    v1 = jnp.max(sel2, axis=1, keepdims=True)
    idx1 = jnp.min(jnp.where(sel2 == v1, e_iota, n_routed), axis=1)

    s0 = jnp.sum(jnp.where(e_iota == idx0[:, None], scores, 0.0), axis=1)
    s1 = jnp.sum(jnp.where(e_iota == idx1[:, None], scores, 0.0), axis=1)
    denom = s0 + s1
    idx_ref[:, 0] = idx0
    idx_ref[:, 1] = idx1
    wts_ref[:, 0] = s0 / denom
    wts_ref[:, 1] = s1 / denom


def _shared_ffn(xf_ref, sw1_ref, sw3_ref, sw2_ref, shared_ref, *, n_shared):
    xb = xf_ref[...].astype(jnp.bfloat16)
    acc = jnp.zeros_like(shared_ref)
    for e in range(n_shared):
        h = _silu(jnp.dot(xb, sw1_ref[e].astype(jnp.bfloat16),
                          preferred_element_type=jnp.float32))
        h = h * jnp.dot(xb, sw3_ref[e].astype(jnp.bfloat16),
                        preferred_element_type=jnp.float32)
        acc = acc + jnp.dot(h.astype(jnp.bfloat16),
                            sw2_ref[e].astype(jnp.bfloat16),
                            preferred_element_type=jnp.float32)
    shared_ref[...] = acc


def _sc_scatter_rows(xf, pos0, pos1):
    t, d = xf.shape
    tk = 2 * t
    rows_w = t // NW
    mesh = plsc.VectorSubcoreMesh(core_axis_name="c", subcore_axis_name="s")

    @functools.partial(
        pl.kernel, mesh=mesh,
        out_type=jax.ShapeDtypeStruct((tk, d), jnp.float32),
        scratch_types=[
            pltpu.VMEM((rows_w,), jnp.int32),
            pltpu.VMEM((rows_w, d), jnp.float32),
            pltpu.SemaphoreType.DMA,
        ],
    )
    def scat(xf_hbm, p0_hbm, p1_hbm, xs_hbm, idx_v, rows_v, sem):
        wid = lax.axis_index("s") * 2 + lax.axis_index("c")
        base = wid * rows_w
        pltpu.sync_copy(xf_hbm.at[pl.ds(base, rows_w)], rows_v)
        pltpu.sync_copy(p0_hbm.at[pl.ds(base, rows_w)], idx_v)
        pltpu.async_copy(rows_v, xs_hbm.at[idx_v], sem).wait()
        pltpu.sync_copy(p1_hbm.at[pl.ds(base, rows_w)], idx_v)
        pltpu.async_copy(rows_v, xs_hbm.at[idx_v], sem).wait()

    return scat(xf, pos0, pos1)


def _grouped_ffn(seg_ref, xs_ref, ws_ref, rw1_ref, rw3_ref, rw2_ref, y_ref,
                 *, n_routed):
    blk = pl.program_id(0)
    base = blk * TBS
    p_iota = base + lax.broadcasted_iota(jnp.int32, (TBS, 1), 0)
    xsb = xs_ref[...].astype(jnp.bfloat16)
    wcol = ws_ref[...]

    y_ref[...] = jnp.zeros_like(y_ref)
    for e in range(n_routed):
        s_e = seg_ref[e]
        e_e = seg_ref[n_routed + e]

        @pl.when((s_e < base + TBS) & (e_e > base))
        def _():
            m = (p_iota >= s_e) & (p_iota < e_e)
            he = _silu(jnp.dot(xsb, rw1_ref[e].astype(jnp.bfloat16),
                               preferred_element_type=jnp.float32))
            he = he * jnp.dot(xsb, rw3_ref[e].astype(jnp.bfloat16),
                              preferred_element_type=jnp.float32)
            he = jnp.where(m, wcol, 0.0) * he
            y_ref[...] += jnp.dot(he.astype(jnp.bfloat16),
                                  rw2_ref[e].astype(jnp.bfloat16),
                                  preferred_element_type=jnp.float32)


def _sc_combine(shared, y, pos0, pos1):
    t, d = shared.shape
    tok_w = t // NW
    nchunk = d // 16
    mesh = plsc.VectorSubcoreMesh(core_axis_name="c", subcore_axis_name="s")

    @functools.partial(
        pl.kernel, mesh=mesh,
        out_type=jax.ShapeDtypeStruct((t, d), jnp.float32),
        scratch_types=[
            pltpu.VMEM((tok_w,), jnp.int32),
            pltpu.VMEM((tok_w, d), jnp.float32),
            pltpu.VMEM((tok_w, d), jnp.float32),
            pltpu.SemaphoreType.DMA,
        ],
    )
    def comb(sh_hbm, y_hbm, p0_hbm, p1_hbm, out_hbm, idx_v, acc_v, g_v, sem):
        wid = lax.axis_index("s") * 2 + lax.axis_index("c")
        base = wid * tok_w
        pltpu.sync_copy(sh_hbm.at[pl.ds(base, tok_w)], acc_v)

        def accumulate():
            def row_body(r, carry):
                for c in range(nchunk):
                    sl = pl.ds(c * 16, 16)
                    acc_v[r, sl] = acc_v[r, sl] + g_v[r, sl]
                return carry
            lax.fori_loop(0, tok_w, row_body, 0)

        pltpu.sync_copy(p0_hbm.at[pl.ds(base, tok_w)], idx_v)
        pltpu.async_copy(y_hbm.at[idx_v], g_v, sem).wait()
        accumulate()
        pltpu.sync_copy(p1_hbm.at[pl.ds(base, tok_w)], idx_v)
        pltpu.async_copy(y_hbm.at[idx_v], g_v, sem).wait()
        accumulate()
        pltpu.sync_copy(acc_v, out_hbm.at[pl.ds(base, tok_w)])

    return comb(shared, y, pos0, pos1)


def kernel(x, shared_w1, shared_w3, shared_w2, routed_w1, routed_w3, routed_w2,
           gate_w, expert_bias):
    b, s, d = x.shape
    t = b * s
    tk = 2 * t
    n_routed = routed_w1.shape[0]
    xf = x.reshape(t, d)

    const = lambda i: (0, 0)
    const3 = lambda i: (0, 0, 0)

    idx, wts = pl.pallas_call(
        functools.partial(_router, n_routed=n_routed),
        grid=(t // TB,),
        in_specs=[
            pl.BlockSpec((TB, d), lambda i: (i, 0)),
            pl.BlockSpec(gate_w.shape, const),
            pl.BlockSpec(expert_bias.shape, lambda i: (0,)),
        ],
        out_specs=[
            pl.BlockSpec((TB, 2), lambda i: (i, 0)),
            pl.BlockSpec((TB, 2), lambda i: (i, 0)),
        ],
        out_shape=[
            jax.ShapeDtypeStruct((t, 2), jnp.int32),
            jax.ShapeDtypeStruct((t, 2), jnp.float32),
        ],
    )(xf, gate_w, expert_bias)

    shared = pl.pallas_call(
        functools.partial(_shared_ffn, n_shared=shared_w1.shape[0]),
        grid=(t // TB,),
        in_specs=[
            pl.BlockSpec((TB, d), lambda i: (i, 0)),
            pl.BlockSpec(shared_w1.shape, const3),
            pl.BlockSpec(shared_w3.shape, const3),
            pl.BlockSpec(shared_w2.shape, const3),
        ],
        out_specs=pl.BlockSpec((TB, d), lambda i: (i, 0)),
        out_shape=jax.ShapeDtypeStruct((t, d), jnp.float32),
    )(xf, shared_w1, shared_w3, shared_w2)

    idx0 = idx[:, 0]
    idx1 = idx[:, 1]
    e_ar = jnp.arange(n_routed, dtype=jnp.int32)
    oh = (idx0[:, None] == e_ar).astype(jnp.int32) + \
         (idx1[:, None] == e_ar).astype(jnp.int32)
    csum = jnp.cumsum(oh, axis=0)
    counts = csum[-1]
    cexcl = csum - oh
    start = jnp.concatenate([jnp.zeros((1,), jnp.int32),
                             jnp.cumsum(counts)[:-1].astype(jnp.int32)])
    seg = jnp.concatenate([start, (start + counts).astype(jnp.int32)])
    slot = start[None, :] + cexcl
    m0 = idx0[:, None] == e_ar
    m1 = idx1[:, None] == e_ar
    pos0 = jnp.sum(jnp.where(m0, slot, 0), axis=1).astype(jnp.int32)
    pos1 = jnp.sum(jnp.where(m1, slot, 0), axis=1).astype(jnp.int32)
    w_sorted = jnp.zeros((tk, 1), jnp.float32)
    w_sorted = w_sorted.at[pos0, 0].set(wts[:, 0])
    w_sorted = w_sorted.at[pos1, 0].set(wts[:, 1])

    xs = _sc_scatter_rows(xf, pos0, pos1)

    y = pl.pallas_call(
        functools.partial(_grouped_ffn, n_routed=n_routed),
        grid_spec=pltpu.PrefetchScalarGridSpec(
            num_scalar_prefetch=1,
            grid=(tk // TBS,),
            in_specs=[
                pl.BlockSpec((TBS, d), lambda i, seg: (i, 0)),
                pl.BlockSpec((TBS, 1), lambda i, seg: (i, 0)),
                pl.BlockSpec(routed_w1.shape, lambda i, seg: (0, 0, 0)),
                pl.BlockSpec(routed_w3.shape, lambda i, seg: (0, 0, 0)),
                pl.BlockSpec(routed_w2.shape, lambda i, seg: (0, 0, 0)),
            ],
            out_specs=pl.BlockSpec((TBS, d), lambda i, seg: (i, 0)),
        ),
        out_shape=jax.ShapeDtypeStruct((tk, d), jnp.float32),
    )(seg, xs, w_sorted, routed_w1, routed_w3, routed_w2)

    out = _sc_combine(shared, y, pos0, pos1)

    return out.reshape(b, s, d), idx.reshape(b, s, 2)


# speedup vs baseline: 2.0332x; 1.7854x over previous
import functools
import jax
import jax.numpy as jnp
from jax import lax
from jax.experimental import pallas as pl

TB = 1024


def _silu(t):
    return t * jax.nn.sigmoid(t)


def _moe_block(xf_ref, sw1_ref, sw3_ref, sw2_ref, rw1_ref, rw3_ref, rw2_ref,
               gw_ref, bias_ref, out_ref, idx_ref, *, n_shared, n_routed):
    xf = xf_ref[...]
    scores = jax.nn.sigmoid(
        jnp.dot(xf, gw_ref[...], preferred_element_type=jnp.float32))
    sel = scores + bias_ref[...]
    e_iota = lax.broadcasted_iota(jnp.int32, sel.shape, 1)

    v0 = jnp.max(sel, axis=1, keepdims=True)
    idx0 = jnp.min(jnp.where(sel == v0, e_iota, n_routed), axis=1)
    sel2 = jnp.where(e_iota == idx0[:, None], -jnp.inf, sel)
    v1 = jnp.max(sel2, axis=1, keepdims=True)
    idx1 = jnp.min(jnp.where(sel2 == v1, e_iota, n_routed), axis=1)

    s0 = jnp.sum(jnp.where(e_iota == idx0[:, None], scores, 0.0), axis=1)
    s1 = jnp.sum(jnp.where(e_iota == idx1[:, None], scores, 0.0), axis=1)
    denom = s0 + s1
    w0 = s0 / denom
    w1 = s1 / denom

    idx_ref[:, 0] = idx0
    idx_ref[:, 1] = idx1

    xb = xf

    acc = jnp.zeros_like(out_ref)
    for e in range(n_shared):
        h = _silu(jnp.dot(xb, sw1_ref[e],
                          preferred_element_type=jnp.float32))
        h = h * jnp.dot(xb, sw3_ref[e],
                        preferred_element_type=jnp.float32)
        acc = acc + jnp.dot(h,
                            sw2_ref[e],
                            preferred_element_type=jnp.float32)

    for e in range(n_routed):
        coef = w0 * (idx0 == e) + w1 * (idx1 == e)
        he = _silu(jnp.dot(xb, rw1_ref[e],
                           preferred_element_type=jnp.float32))
        he = he * jnp.dot(xb, rw3_ref[e],
                          preferred_element_type=jnp.float32)
        acc = acc + jnp.dot(coef[:, None] * he,
                            rw2_ref[e],
                            preferred_element_type=jnp.float32)

    out_ref[...] = acc


def kernel(x, shared_w1, shared_w3, shared_w2, routed_w1, routed_w3, routed_w2,
           gate_w, expert_bias):
    b, s, d = x.shape
    t = b * s
    n_shared = shared_w1.shape[0]
    n_routed = routed_w1.shape[0]
    xf = x.reshape(t, d)

    const = lambda i: (0, 0)
    const3 = lambda i: (0, 0, 0)

    out, idx = pl.pallas_call(
        functools.partial(_moe_block, n_shared=n_shared, n_routed=n_routed),
        grid=(t // TB,),
        in_specs=[
            pl.BlockSpec((TB, d), lambda i: (i, 0)),
            pl.BlockSpec(shared_w1.shape, const3),
            pl.BlockSpec(shared_w3.shape, const3),
            pl.BlockSpec(shared_w2.shape, const3),
            pl.BlockSpec(routed_w1.shape, const3),
            pl.BlockSpec(routed_w3.shape, const3),
            pl.BlockSpec(routed_w2.shape, const3),
            pl.BlockSpec(gate_w.shape, const),
            pl.BlockSpec(expert_bias.shape, lambda i: (0,)),
        ],
        out_specs=[
            pl.BlockSpec((TB, d), lambda i: (i, 0)),
            pl.BlockSpec((TB, 2), lambda i: (i, 0)),
        ],
        out_shape=[
            jax.ShapeDtypeStruct((t, d), jnp.float32),
            jax.ShapeDtypeStruct((t, 2), jnp.int32),
        ],
    )(xf, shared_w1, shared_w3, shared_w2, routed_w1, routed_w3, routed_w2,
      gate_w, expert_bias)

    return out.reshape(b, s, d), idx.reshape(b, s, 2)
